# Initial kernel scaffold; baseline (speedup 1.0000x reference)
import jax
import jax.numpy as jnp
from jax.experimental import pallas as pl


def kernel(p, x, Wq, bq, Wk, bk, Wv, bv, pe_W1, pe_bn_g, pe_bn_b, pe_W2, pe_cb, at_bn1_g, at_bn1_b, at_W1, at_bn2_g, at_bn2_b, at_W2, at_cb):
    raise NotImplementedError("write your pallas kernel here")


# speedup vs baseline: 7.4064x; 7.4064x over previous
import functools

import jax
import jax.numpy as jnp
from jax import lax
from jax.experimental import pallas as pl
from jax.experimental.pallas import tpu as pltpu
from jax.experimental.pallas import tpu_sc as plsc

_B, _N, _K = 4, 4096, 16
_C = 128
_ATTN, _SHARED = 16, 8
_PD = 16
_RK = 256
_TQ = 512
_TP = 512
_CNT = _B * _N * _K
_EPS = 1e-5

_NC, _NS = 2, 16
_NW = _NC * _NS
_PER_W = _CNT // _NW
_CH = 256
_NCH = _PER_W // _CH


def _f32(x):
    return x.astype(jnp.float32)


def _qkv_kernel(x_ref, wq_ref, bq_ref, wk_ref, bk_ref, wv_ref, bv_ref,
                q_ref, kv_ref):
    xb = x_ref[0]

    def proj(w_ref, b_ref):
        return lax.dot_general(xb, w_ref[...], (((0,), (1,)), ((), ())),
                               preferred_element_type=jnp.float32) + b_ref[...]

    q_ref[0] = proj(wq_ref, bq_ref)
    kv_ref[0] = jnp.concatenate([proj(wk_ref, bk_ref), proj(wv_ref, bv_ref)],
                                axis=1)


def _run_qkv(x, wq, bq, wk, bk, wv, bv):
    nt = _N // _TQ
    wspec = pl.BlockSpec((_C, _C), lambda b, i: (0, 0))
    bspec = pl.BlockSpec((1, _C), lambda b, i: (0, 0))
    q, kv = pl.pallas_call(
        _qkv_kernel,
        grid=(_B, nt),
        in_specs=[
            pl.BlockSpec((1, _C, _TQ), lambda b, i: (b, 0, i)),
            wspec, bspec, wspec, bspec, wspec, bspec,
        ],
        out_specs=[
            pl.BlockSpec((1, _TQ, _C), lambda b, i: (b, i, 0)),
            pl.BlockSpec((1, _TQ, 2 * _C), lambda b, i: (b, i, 0)),
        ],
        out_shape=[
            jax.ShapeDtypeStruct((_B, _N, _C), jnp.float32),
            jax.ShapeDtypeStruct((_B, _N, 2 * _C), jnp.float32),
        ],
    )(x, wq, bq.reshape(1, _C), wk, bk.reshape(1, _C), wv, bv.reshape(1, _C))
    return q.reshape(_B * _N, _C), kv.reshape(_B * _N, 2 * _C)


def _knn_kernel(pt_ref, pf_ref, pft_ref, idx_ref, rel_ref):
    b = pl.program_id(0)
    pt = pt_ref[0]
    pf = pf_ref[0]
    pft = pft_ref[0]
    sqt = (pt[:, 0:1] * pt[:, 0:1] + pt[:, 1:2] * pt[:, 1:2]) \
        + pt[:, 2:3] * pt[:, 2:3]
    sqf = (pft[0:1, :] * pft[0:1, :] + pft[1:2, :] * pft[1:2, :]) \
        + pft[2:3, :] * pft[2:3, :]
    dot = lax.dot_general(pt.astype(jnp.bfloat16), pf.astype(jnp.bfloat16),
                          (((1,), (1,)), ((), ())),
                          preferred_element_type=jnp.float32)
    d = (sqt + sqf) - 2.0 * dot

    iota = lax.broadcasted_iota(jnp.int32, d.shape, 1)
    big_i = jnp.int32(2 ** 30)
    inf = jnp.float32(3e38)
    px = pft[0:1, :]
    py = pft[1:2, :]
    pz = pft[2:3, :]
    otx = pt[:, 0:1]
    oty = pt[:, 1:2]
    otz = pt[:, 2:3]
    zp13 = jnp.zeros((pt.shape[0], _PD - 3), jnp.float32)
    cols = []
    rels = []
    for _ in range(_K):
        m = jnp.min(d, axis=1, keepdims=True)
        cand = jnp.where(d == m, iota, big_i)
        ij = jnp.min(cand, axis=1, keepdims=True)
        cols.append(ij)
        sel = iota == ij
        cx = jnp.sum(jnp.where(sel, px, 0.0), axis=1, keepdims=True)
        cy = jnp.sum(jnp.where(sel, py, 0.0), axis=1, keepdims=True)
        cz = jnp.sum(jnp.where(sel, pz, 0.0), axis=1, keepdims=True)
        rels.append(jnp.concatenate(
            [cx - otx, cy - oty, cz - otz, zp13], axis=1))
        d = jnp.where(sel, inf, d)
    idx_ref[0] = jnp.concatenate(cols, axis=1) + b * _N
    rel_ref[...] = jnp.stack(rels, axis=1).reshape(pt.shape[0] * _K, _PD)


def _run_knn(ppad3, pt8):
    nt = _N // _RK
    idx, rel = pl.pallas_call(
        _knn_kernel,
        grid=(_B, nt),
        in_specs=[
            pl.BlockSpec((1, _RK, _PD), lambda b, i: (b, i, 0)),
            pl.BlockSpec((1, _N, _PD), lambda b, i: (b, 0, 0)),
            pl.BlockSpec((1, 8, _N), lambda b, i: (b, 0, 0)),
        ],
        out_specs=[
            pl.BlockSpec((1, _RK, _K), lambda b, i: (b, i, 0)),
            pl.BlockSpec((_RK * _K, _PD), lambda b, i: (b * nt + i, 0)),
        ],
        out_shape=[
            jax.ShapeDtypeStruct((_B, _N, _K), jnp.int32),
            jax.ShapeDtypeStruct((_CNT, _PD), jnp.float32),
        ],
    )(ppad3, ppad3, pt8)
    return idx.reshape(_CNT), rel


def _gather_sc(kv, idx_flat):
    mesh = plsc.VectorSubcoreMesh(core_axis_name="c", subcore_axis_name="s")

    @functools.partial(
        pl.kernel,
        mesh=mesh,
        out_type=jax.ShapeDtypeStruct((_CNT, 2 * _C), jnp.float32),
        scratch_types=[
            pltpu.VMEM((_CH,), jnp.int32),
            pltpu.VMEM((_CH, 2 * _C), jnp.float32),
            pltpu.SemaphoreType.DMA,
        ],
    )
    def gk(kv_hbm, idx_hbm, okv_hbm, idx_v, kv_v, s1):
        wid = lax.axis_index("s") * _NC + lax.axis_index("c")
        base = wid * _PER_W

        @pl.loop(0, _NCH)
        def _(c):
            off = base + c * _CH
            pltpu.sync_copy(idx_hbm.at[pl.ds(off, _CH)], idx_v)
            pltpu.async_copy(kv_hbm.at[idx_v], kv_v, s1).wait()
            pltpu.sync_copy(kv_v, okv_hbm.at[pl.ds(off, _CH)])

    return gk(kv, idx_flat)


def _rep_rows(t, k):
    tt, ll = t.shape
    return jnp.broadcast_to(t.reshape(tt, 1, ll), (tt, k, ll)).reshape(tt * k, ll)


def _p0_kernel(rel_ref, s1_ref, s2_ref):
    rel = rel_ref[...]

    @pl.when(pl.program_id(0) == 0)
    def _():
        s1_ref[...] = jnp.zeros_like(s1_ref)
        s2_ref[...] = jnp.zeros_like(s2_ref)

    s1_ref[...] += jnp.sum(rel, axis=0, keepdims=True)
    s2_ref[...] += lax.dot_general(rel, rel, (((0,), (0,)), ((), ())),
                                   preferred_element_type=jnp.float32)


def _nr_from(rel, w1p, spe, tpe, w2pt, pecb):
    h = lax.dot_general(rel, w1p, (((1,), (1,)), ((), ())),
                        preferred_element_type=jnp.float32)
    hp = jnp.maximum(h * spe + tpe, 0.0)
    return lax.dot_general(hp, w2pt, (((1,), (0,)), ((), ())),
                           preferred_element_type=jnp.float32) + pecb


def _p1_kernel(nk_ref, rel_ref, q_ref, w1p_ref, spe_ref, tpe_ref,
               w2pt_ref, pecb_ref, s1_ref, s2_ref):
    nr = _nr_from(rel_ref[...], w1p_ref[...], spe_ref[...],
                  tpe_ref[...], w2pt_ref[...], pecb_ref[...])
    a = _rep_rows(q_ref[...], _K) - nk_ref[...] + nr

    @pl.when(pl.program_id(0) == 0)
    def _():
        s1_ref[...] = jnp.zeros_like(s1_ref)
        s2_ref[...] = jnp.zeros_like(s2_ref)

    s1_ref[...] += jnp.sum(a, axis=0, keepdims=True)
    s2_ref[...] += jnp.sum(a * a, axis=0, keepdims=True)


def _p2_kernel(nk_ref, rel_ref, q_ref, w1p_ref, spe_ref, tpe_ref,
               w2pt_ref, pecb_ref, s1a_ref, t1a_ref, aw1_ref,
               s1_ref, s2_ref):
    nr = _nr_from(rel_ref[...], w1p_ref[...], spe_ref[...],
                  tpe_ref[...], w2pt_ref[...], pecb_ref[...])
    a = _rep_rows(q_ref[...], _K) - nk_ref[...] + nr
    ap = jnp.maximum(a * s1a_ref[...] + t1a_ref[...], 0.0)
    a1 = lax.dot_general(ap, aw1_ref[...], (((1,), (1,)), ((), ())),
                         preferred_element_type=jnp.float32)

    @pl.when(pl.program_id(0) == 0)
    def _():
        s1_ref[...] = jnp.zeros_like(s1_ref)
        s2_ref[...] = jnp.zeros_like(s2_ref)

    s1_ref[...] += jnp.sum(a1, axis=0, keepdims=True)
    s2_ref[...] += jnp.sum(a1 * a1, axis=0, keepdims=True)


def _p3_kernel(kv_ref, rel_ref, q_ref, w1p_ref, spe_ref, tpe_ref,
               w2pt_ref, pecb_ref, s1a_ref, t1a_ref, aw1_ref,
               s2a_ref, t2a_ref, aw2_ref, acb_ref, exp_ref, y_ref):
    nr = _nr_from(rel_ref[...], w1p_ref[...], spe_ref[...],
                  tpe_ref[...], w2pt_ref[...], pecb_ref[...])
    kvb = kv_ref[...]
    nk = kvb[:, 0:_C]
    nv = kvb[:, _C:2 * _C]
    a = _rep_rows(q_ref[...], _K) - nk + nr
    ap = jnp.maximum(a * s1a_ref[...] + t1a_ref[...], 0.0)
    a1 = lax.dot_general(ap, aw1_ref[...], (((1,), (1,)), ((), ())),
                         preferred_element_type=jnp.float32)
    a1p = jnp.maximum(a1 * s2a_ref[...] + t2a_ref[...], 0.0)
    a2 = lax.dot_general(a1p, aw2_ref[...], (((1,), (1,)), ((), ())),
                         preferred_element_type=jnp.float32) + acb_ref[...]
    a3 = a2.reshape(_TP, _K, _ATTN)
    mx = jnp.max(a3, axis=1, keepdims=True)
    e = jnp.exp(a3 - mx)
    w = e / jnp.sum(e, axis=1, keepdims=True)
    wf = w.reshape(_TP * _K, _ATTN)
    w128 = lax.dot_general(wf, exp_ref[...], (((1,), (0,)), ((), ())),
                           preferred_element_type=jnp.float32)
    contrib = w128 * (nv + nr)
    y_ref[...] = jnp.sum(contrib.reshape(_TP, _K, _C), axis=1)


def kernel(p, x, Wq, bq, Wk, bk, Wv, bv, pe_W1, pe_bn_g, pe_bn_b, pe_W2,
           pe_cb, at_bn1_g, at_bn1_b, at_W1, at_bn2_g, at_bn2_b, at_W2,
           at_cb):
    f = _f32
    p = f(p)
    x = f(x)

    pflat = p.reshape(_B * _N, 3)
    ppad = jnp.concatenate(
        [pflat, jnp.zeros((_B * _N, _PD - 3), jnp.float32)], axis=1)
    ppad3 = ppad.reshape(_B, _N, _PD)

    pt8 = jnp.concatenate(
        [p.transpose(0, 2, 1), jnp.zeros((_B, 5, _N), jnp.float32)], axis=1)

    q, kv = _run_qkv(x, f(Wq), f(bq), f(Wk), f(bk), f(Wv), f(bv))
    idx_flat, rel_g = _run_knn(ppad3, pt8)
    kv_g = _gather_sc(kv, idx_flat)

    grid = (_CNT // (_TP * _K),)
    rel_spec = pl.BlockSpec((_TP * _K, _PD), lambda i: (i, 0))
    q_spec = pl.BlockSpec((_TP, _C), lambda i: (i, 0))
    nk_spec = pl.BlockSpec((_TP * _K, _C), lambda i: (i, 0))
    kv_spec = pl.BlockSpec((_TP * _K, 2 * _C), lambda i: (i, 0))

    def const_spec(shape):
        nd = len(shape)
        return pl.BlockSpec(shape, lambda i, _n=nd: (0,) * _n)

    acc_spec = const_spec

    s1r, s2r = pl.pallas_call(
        _p0_kernel,
        grid=grid,
        in_specs=[rel_spec],
        out_specs=[acc_spec((1, _PD)), acc_spec((_PD, _PD))],
        out_shape=[jax.ShapeDtypeStruct((1, _PD), jnp.float32),
                   jax.ShapeDtypeStruct((_PD, _PD), jnp.float32)],
    )(rel_g)

    cnt = jnp.float32(_CNT)
    w1p = jnp.zeros((_PD, _PD), jnp.float32).at[0:3, 0:3].set(f(pe_W1))
    mean_rel = s1r / cnt
    cov = s2r / cnt - mean_rel.T @ mean_rel
    mh = mean_rel @ w1p.T
    vh = jnp.sum((w1p @ cov) * w1p, axis=1).reshape(1, _PD)
    g16 = jnp.zeros((1, _PD), jnp.float32).at[0, 0:3].set(f(pe_bn_g))
    b16 = jnp.zeros((1, _PD), jnp.float32).at[0, 0:3].set(f(pe_bn_b))
    spe = g16 * lax.rsqrt(vh + _EPS)
    tpe = b16 - mh * spe
    w2pt = jnp.zeros((_PD, _C), jnp.float32).at[0:3, :].set(f(pe_W2).T)
    pecb = f(pe_cb).reshape(1, _C)

    wconsts = [const_spec((_PD, _PD)), const_spec((1, _PD)),
               const_spec((1, _PD)), const_spec((_PD, _C)),
               const_spec((1, _C))]
    wvals = (w1p, spe, tpe, w2pt, pecb)

    s1a_s, s2a_s = pl.pallas_call(
        _p1_kernel,
        grid=grid,
        in_specs=[nk_spec, rel_spec, q_spec] + wconsts,
        out_specs=[acc_spec((1, _C)), acc_spec((1, _C))],
        out_shape=[jax.ShapeDtypeStruct((1, _C), jnp.float32),
                   jax.ShapeDtypeStruct((1, _C), jnp.float32)],
    )(kv_g, rel_g, q, *wvals)

    mean1 = s1a_s / cnt
    var1 = s2a_s / cnt - mean1 * mean1
    s1a = f(at_bn1_g).reshape(1, _C) * lax.rsqrt(var1 + _EPS)
    t1a = f(at_bn1_b).reshape(1, _C) - mean1 * s1a
    aw1 = f(at_W1)

    s1b_s, s2b_s = pl.pallas_call(
        _p2_kernel,
        grid=grid,
        in_specs=[nk_spec, rel_spec, q_spec] + wconsts +
                 [const_spec((1, _C)), const_spec((1, _C)),
                  const_spec((_ATTN, _C))],
        out_specs=[acc_spec((1, _ATTN)), acc_spec((1, _ATTN))],
        out_shape=[jax.ShapeDtypeStruct((1, _ATTN), jnp.float32),
                   jax.ShapeDtypeStruct((1, _ATTN), jnp.float32)],
    )(kv_g, rel_g, q, *wvals, s1a, t1a, aw1)

    mean2 = s1b_s / cnt
    var2 = s2b_s / cnt - mean2 * mean2
    s2a = f(at_bn2_g).reshape(1, _ATTN) * lax.rsqrt(var2 + _EPS)
    t2a = f(at_bn2_b).reshape(1, _ATTN) - mean2 * s2a
    aw2 = f(at_W2)
    acb = f(at_cb).reshape(1, _ATTN)
    expm = (jnp.arange(_ATTN)[:, None] ==
            (jnp.arange(_C)[None, :] // _SHARED)).astype(jnp.float32)

    y = pl.pallas_call(
        _p3_kernel,
        grid=grid,
        in_specs=[kv_spec, rel_spec, q_spec] + wconsts +
                 [const_spec((1, _C)), const_spec((1, _C)),
                  const_spec((_ATTN, _C)), const_spec((1, _ATTN)),
                  const_spec((1, _ATTN)), const_spec((_ATTN, _ATTN)),
                  const_spec((1, _ATTN)), const_spec((_ATTN, _C))],
        out_specs=pl.BlockSpec((_TP, _C), lambda i: (i, 0)),
        out_shape=jax.ShapeDtypeStruct((_B * _N, _C), jnp.float32),
    )(kv_g, rel_g, q, *wvals, s1a, t1a, aw1, s2a, t2a, aw2, acb, expm)

    return y.reshape(_B, _N, _C).transpose(0, 2, 1)


# speedup vs baseline: 13.9034x; 1.8772x over previous
import functools

import jax
import jax.numpy as jnp
from jax import lax
from jax.experimental import pallas as pl
from jax.experimental.pallas import tpu as pltpu
from jax.experimental.pallas import tpu_sc as plsc

_B, _N, _K = 4, 4096, 16
_C = 128
_ATTN, _SHARED = 16, 8
_PD = 16
_RK = 256
_TQ = 512
_TP = 512
_CNT = _B * _N * _K
_EPS = 1e-5

_NC, _NS = 2, 16
_NW = _NC * _NS
_PER_W = _CNT // _NW
_CH = 256
_NCH = _PER_W // _CH


def _f32(x):
    return x.astype(jnp.float32)


def _qkv_kernel(x_ref, wq_ref, bq_ref, wk_ref, bk_ref, wv_ref,
                bv_ref, q_ref, kv_ref):
    xb = x_ref[0]

    def proj(w_ref, b_ref):
        return lax.dot_general(xb, w_ref[...], (((0,), (1,)), ((), ())),
                               preferred_element_type=jnp.float32) + b_ref[...]

    q_ref[0] = proj(wq_ref, bq_ref)
    kv_ref[0] = jnp.concatenate(
        [proj(wk_ref, bk_ref), proj(wv_ref, bv_ref)], axis=1)


def _run_qkv(x, wq, bq, wk, bk, wv, bv):
    nt = _N // _TQ
    wspec = pl.BlockSpec((_C, _C), lambda b, i: (0, 0))
    bspec = pl.BlockSpec((1, _C), lambda b, i: (0, 0))
    q, kv = pl.pallas_call(
        _qkv_kernel,
        grid=(_B, nt),
        in_specs=[
            pl.BlockSpec((1, _C, _TQ), lambda b, i: (b, 0, i)),
            wspec, bspec, wspec, bspec, wspec, bspec,
        ],
        out_specs=[
            pl.BlockSpec((1, _TQ, _C), lambda b, i: (b, i, 0)),
            pl.BlockSpec((1, _TQ, 2 * _C), lambda b, i: (b, i, 0)),
        ],
        out_shape=[
            jax.ShapeDtypeStruct((_B, _N, _C), jnp.float32),
            jax.ShapeDtypeStruct((_B, _N, 2 * _C), jnp.float32),
        ],
    )(x, wq, bq.reshape(1, _C), wk, bk.reshape(1, _C), wv,
      bv.reshape(1, _C))
    return q.reshape(_B * _N, _C), kv.reshape(_B * _N, 2 * _C)


def _knn_kernel(pt_ref, pf_ref, pft_ref, idx_ref):
    b = pl.program_id(0)
    pt = pt_ref[0]
    pf = pf_ref[0]
    pft = pft_ref[0]
    sqt = (pt[:, 0:1] * pt[:, 0:1] + pt[:, 1:2] * pt[:, 1:2]) \
        + pt[:, 2:3] * pt[:, 2:3]
    sqf = (pft[0:1, :] * pft[0:1, :] + pft[1:2, :] * pft[1:2, :]) \
        + pft[2:3, :] * pft[2:3, :]
    dot = lax.dot_general(pt.astype(jnp.bfloat16), pf.astype(jnp.bfloat16),
                          (((1,), (1,)), ((), ())),
                          preferred_element_type=jnp.float32)
    d = (sqt + sqf) - 2.0 * dot

    iota = lax.broadcasted_iota(jnp.int32, d.shape, 1)
    big_i = jnp.int32(2 ** 30)
    inf = jnp.float32(3e38)
    cols = []
    for _ in range(_K):
        m = jnp.min(d, axis=1, keepdims=True)
        cand = jnp.where(d == m, iota, big_i)
        ij = jnp.min(cand, axis=1, keepdims=True)
        cols.append(ij)
        d = jnp.where(cand == ij, inf, d)
    idx_ref[0] = jnp.concatenate(cols, axis=1) + b * _N


def _run_knn(ppad3, pt8):
    nt = _N // _RK
    idx = pl.pallas_call(
        _knn_kernel,
        grid=(_B, nt),
        in_specs=[
            pl.BlockSpec((1, _RK, _PD), lambda b, i: (b, i, 0)),
            pl.BlockSpec((1, _N, _PD), lambda b, i: (b, 0, 0)),
            pl.BlockSpec((1, 8, _N), lambda b, i: (b, 0, 0)),
        ],
        out_specs=pl.BlockSpec((1, _RK, _K), lambda b, i: (b, i, 0)),
        out_shape=jax.ShapeDtypeStruct((_B, _N, _K), jnp.int32),
        compiler_params=pltpu.CompilerParams(
            dimension_semantics=("parallel", "parallel")),
    )(ppad3, ppad3, pt8)
    return idx.reshape(_CNT)


def _gather_rows(table, idx_flat, width):
    mesh = plsc.VectorSubcoreMesh(core_axis_name="c", subcore_axis_name="s")

    @functools.partial(
        pl.kernel,
        mesh=mesh,
        out_type=jax.ShapeDtypeStruct((_CNT, width), jnp.float32),
        scratch_types=[
            pltpu.VMEM((_CH,), jnp.int32),
            pltpu.VMEM((_CH, width), jnp.float32),
            pltpu.SemaphoreType.DMA,
        ],
    )
    def gk(tab_hbm, idx_hbm, out_hbm, idx_v, row_v, s1):
        wid = lax.axis_index("s") * _NC + lax.axis_index("c")
        base = wid * _PER_W

        @pl.loop(0, _NCH)
        def _(c):
            off = base + c * _CH
            pltpu.sync_copy(idx_hbm.at[pl.ds(off, _CH)], idx_v)
            pltpu.async_copy(tab_hbm.at[idx_v], row_v, s1).wait()
            pltpu.sync_copy(row_v, out_hbm.at[pl.ds(off, _CH)])

    return gk(table, idx_flat)


def _rep_rows(t, k):
    tt, ll = t.shape
    return jnp.broadcast_to(t.reshape(tt, 1, ll), (tt, k, ll)).reshape(tt * k, ll)


def _p0_kernel(np_ref, pp_ref, s1_ref, s2_ref, rel_ref):
    rel = np_ref[:, 0:_PD] - _rep_rows(pp_ref[...], _K)
    rel_ref[...] = rel

    @pl.when(pl.program_id(0) == 0)
    def _():
        s1_ref[...] = jnp.zeros_like(s1_ref)
        s2_ref[...] = jnp.zeros_like(s2_ref)

    s1_ref[...] += jnp.sum(rel, axis=0, keepdims=True)
    s2_ref[...] += lax.dot_general(rel, rel, (((0,), (0,)), ((), ())),
                                   preferred_element_type=jnp.float32)


def _nr_from(rel, w1p, spe, tpe, w2pt, pecb):
    h = lax.dot_general(rel, w1p, (((1,), (1,)), ((), ())),
                        preferred_element_type=jnp.float32)
    hp = jnp.maximum(h * spe + tpe, 0.0)
    return lax.dot_general(hp, w2pt, (((1,), (0,)), ((), ())),
                           preferred_element_type=jnp.float32) + pecb


def _p1_kernel(nk_ref, rel_ref, q_ref, w1p_ref, spe_ref, tpe_ref,
               w2pt_ref, pecb_ref, s1_ref, s2_ref):
    nr = _nr_from(rel_ref[...], w1p_ref[...],
                  spe_ref[...], tpe_ref[...], w2pt_ref[...], pecb_ref[...])
    a = _rep_rows(q_ref[...], _K) - nk_ref[...] + nr

    @pl.when(pl.program_id(0) == 0)
    def _():
        s1_ref[...] = jnp.zeros_like(s1_ref)
        s2_ref[...] = jnp.zeros_like(s2_ref)

    s1_ref[...] += jnp.sum(a, axis=0, keepdims=True)
    s2_ref[...] += jnp.sum(a * a, axis=0, keepdims=True)


def _p2_kernel(nk_ref, rel_ref, q_ref, w1p_ref, spe_ref, tpe_ref,
               w2pt_ref, pecb_ref, s1a_ref, t1a_ref, aw1_ref,
               s1_ref, s2_ref):
    nr = _nr_from(rel_ref[...], w1p_ref[...],
                  spe_ref[...], tpe_ref[...], w2pt_ref[...], pecb_ref[...])
    a = _rep_rows(q_ref[...], _K) - nk_ref[...] + nr
    ap = jnp.maximum(a * s1a_ref[...] + t1a_ref[...], 0.0)
    a1 = lax.dot_general(ap, aw1_ref[...], (((1,), (1,)), ((), ())),
                         preferred_element_type=jnp.float32)

    @pl.when(pl.program_id(0) == 0)
    def _():
        s1_ref[...] = jnp.zeros_like(s1_ref)
        s2_ref[...] = jnp.zeros_like(s2_ref)

    s1_ref[...] += jnp.sum(a1, axis=0, keepdims=True)
    s2_ref[...] += jnp.sum(a1 * a1, axis=0, keepdims=True)


def _p3_kernel(kv_ref, rel_ref, q_ref, w1p_ref, spe_ref, tpe_ref,
               w2pt_ref, pecb_ref, s1a_ref, t1a_ref, aw1_ref,
               s2a_ref, t2a_ref, aw2_ref, acb_ref, exp_ref, y_ref):
    nr = _nr_from(rel_ref[...], w1p_ref[...],
                  spe_ref[...], tpe_ref[...], w2pt_ref[...], pecb_ref[...])
    kvb = kv_ref[...]
    nk = kvb[:, 0:_C]
    nv = kvb[:, _C:2 * _C]
    a = _rep_rows(q_ref[...], _K) - nk + nr
    ap = jnp.maximum(a * s1a_ref[...] + t1a_ref[...], 0.0)
    a1 = lax.dot_general(ap, aw1_ref[...], (((1,), (1,)), ((), ())),
                         preferred_element_type=jnp.float32)
    a1p = jnp.maximum(a1 * s2a_ref[...] + t2a_ref[...], 0.0)
    a2 = lax.dot_general(a1p, aw2_ref[...], (((1,), (1,)), ((), ())),
                         preferred_element_type=jnp.float32) + acb_ref[...]
    a3 = a2.reshape(_TP, _K, _ATTN)
    mx = jnp.max(a3, axis=1, keepdims=True)
    e = jnp.exp(a3 - mx)
    w = e / jnp.sum(e, axis=1, keepdims=True)
    wf = w.reshape(_TP * _K, _ATTN)
    w128 = lax.dot_general(wf, exp_ref[...], (((1,), (0,)), ((), ())),
                           preferred_element_type=jnp.float32)
    contrib = w128 * (nv + nr)
    y_ref[...] = jnp.sum(contrib.reshape(_TP, _K, _C), axis=1)


def kernel(p, x, Wq, bq, Wk, bk, Wv, bv, pe_W1, pe_bn_g, pe_bn_b, pe_W2,
           pe_cb, at_bn1_g, at_bn1_b, at_W1, at_bn2_g, at_bn2_b, at_W2,
           at_cb):
    f = _f32
    p = f(p)
    x = f(x)

    pflat = p.reshape(_B * _N, 3)
    ppad = jnp.concatenate(
        [pflat, jnp.zeros((_B * _N, _PD - 3), jnp.float32)], axis=1)
    ppad3 = ppad.reshape(_B, _N, _PD)
    ppad128 = jnp.concatenate(
        [pflat, jnp.zeros((_B * _N, _C - 3), jnp.float32)], axis=1)

    pt8 = jnp.concatenate(
        [p.transpose(0, 2, 1), jnp.zeros((_B, 5, _N), jnp.float32)], axis=1)

    idx_flat = _run_knn(ppad3, pt8)
    p_g = _gather_rows(ppad128, idx_flat, _C)
    q, kvp = _run_qkv(x, f(Wq), f(bq), f(Wk), f(bk), f(Wv), f(bv))
    kv_g = _gather_rows(kvp, idx_flat, 2 * _C)

    grid = (_CNT // (_TP * _K),)
    np_spec = pl.BlockSpec((_TP * _K, _C), lambda i: (i, 0))
    pp_spec = pl.BlockSpec((_TP, _PD), lambda i: (i, 0))
    relc_spec = pl.BlockSpec((_TP * _K, _PD), lambda i: (i, 0))
    q_spec = pl.BlockSpec((_TP, _C), lambda i: (i, 0))
    nk_spec = pl.BlockSpec((_TP * _K, _C), lambda i: (i, 0))
    kv_spec = pl.BlockSpec((_TP * _K, 2 * _C), lambda i: (i, 0))

    def const_spec(shape):
        nd = len(shape)
        return pl.BlockSpec(shape, lambda i, _n=nd: (0,) * _n)

    acc_spec = const_spec

    s1r, s2r, rel_c = pl.pallas_call(
        _p0_kernel,
        grid=grid,
        in_specs=[np_spec, pp_spec],
        out_specs=[acc_spec((1, _PD)), acc_spec((_PD, _PD)), relc_spec],
        out_shape=[jax.ShapeDtypeStruct((1, _PD), jnp.float32),
                   jax.ShapeDtypeStruct((_PD, _PD), jnp.float32),
                   jax.ShapeDtypeStruct((_CNT, _PD), jnp.float32)],
    )(p_g, ppad)

    cnt = jnp.float32(_CNT)
    w1p = jnp.zeros((_PD, _PD), jnp.float32).at[0:3, 0:3].set(f(pe_W1))
    mean_rel = s1r / cnt
    cov = s2r / cnt - mean_rel.T @ mean_rel
    mh = mean_rel @ w1p.T
    vh = jnp.sum((w1p @ cov) * w1p, axis=1).reshape(1, _PD)
    g16 = jnp.zeros((1, _PD), jnp.float32).at[0, 0:3].set(f(pe_bn_g))
    b16 = jnp.zeros((1, _PD), jnp.float32).at[0, 0:3].set(f(pe_bn_b))
    spe = g16 * lax.rsqrt(vh + _EPS)
    tpe = b16 - mh * spe
    w2pt = jnp.zeros((_PD, _C), jnp.float32).at[0:3, :].set(f(pe_W2).T)
    pecb = f(pe_cb).reshape(1, _C)

    wconsts = [const_spec((_PD, _PD)), const_spec((1, _PD)),
               const_spec((1, _PD)), const_spec((_PD, _C)),
               const_spec((1, _C))]
    wvals = (w1p, spe, tpe, w2pt, pecb)

    s1a_s, s2a_s = pl.pallas_call(
        _p1_kernel,
        grid=grid,
        in_specs=[nk_spec, relc_spec, q_spec] + wconsts,
        out_specs=[acc_spec((1, _C)), acc_spec((1, _C))],
        out_shape=[jax.ShapeDtypeStruct((1, _C), jnp.float32),
                   jax.ShapeDtypeStruct((1, _C), jnp.float32)],
    )(kv_g, rel_c, q, *wvals)

    mean1 = s1a_s / cnt
    var1 = s2a_s / cnt - mean1 * mean1
    s1a = f(at_bn1_g).reshape(1, _C) * lax.rsqrt(var1 + _EPS)
    t1a = f(at_bn1_b).reshape(1, _C) - mean1 * s1a
    aw1 = f(at_W1)

    s1b_s, s2b_s = pl.pallas_call(
        _p2_kernel,
        grid=grid,
        in_specs=[nk_spec, relc_spec, q_spec] + wconsts +
                 [const_spec((1, _C)), const_spec((1, _C)),
                  const_spec((_ATTN, _C))],
        out_specs=[acc_spec((1, _ATTN)), acc_spec((1, _ATTN))],
        out_shape=[jax.ShapeDtypeStruct((1, _ATTN), jnp.float32),
                   jax.ShapeDtypeStruct((1, _ATTN), jnp.float32)],
    )(kv_g, rel_c, q, *wvals, s1a, t1a, aw1)

    mean2 = s1b_s / cnt
    var2 = s2b_s / cnt - mean2 * mean2
    s2a = f(at_bn2_g).reshape(1, _ATTN) * lax.rsqrt(var2 + _EPS)
    t2a = f(at_bn2_b).reshape(1, _ATTN) - mean2 * s2a
    aw2 = f(at_W2)
    acb = f(at_cb).reshape(1, _ATTN)
    expm = (jnp.arange(_ATTN)[:, None] ==
            (jnp.arange(_C)[None, :] // _SHARED)).astype(jnp.float32)

    y = pl.pallas_call(
        _p3_kernel,
        grid=grid,
        in_specs=[kv_spec, relc_spec, q_spec] + wconsts +
                 [const_spec((1, _C)), const_spec((1, _C)),
                  const_spec((_ATTN, _C)), const_spec((1, _ATTN)),
                  const_spec((1, _ATTN)), const_spec((_ATTN, _ATTN)),
                  const_spec((1, _ATTN)), const_spec((_ATTN, _C))],
        out_specs=pl.BlockSpec((_TP, _C), lambda i: (i, 0)),
        out_shape=jax.ShapeDtypeStruct((_B * _N, _C), jnp.float32),
    )(kv_g, rel_c, q, *wvals, s1a, t1a, aw1, s2a, t2a, aw2, acb, expm)

    return y.reshape(_B, _N, _C).transpose(0, 2, 1)


# speedup vs baseline: 14.5861x; 1.0491x over previous
import functools

import jax
import jax.numpy as jnp
from jax import lax
from jax.experimental import pallas as pl
from jax.experimental.pallas import tpu as pltpu
from jax.experimental.pallas import tpu_sc as plsc

_B, _N, _K = 4, 4096, 16
_C = 128
_ATTN, _SHARED = 16, 8
_PD = 16
_RK = 256
_TQ = 512
_TP = 512
_CNT = _B * _N * _K
_EPS = 1e-5

_NC, _NS = 2, 16
_NW = _NC * _NS
_PER_W = _CNT // _NW
_CH = 256
_NCH = _PER_W // _CH


def _f32(x):
    return x.astype(jnp.float32)


def _pack_kv(k, v):
    ku = lax.bitcast_convert_type(
        k.astype(jnp.bfloat16).astype(jnp.float32), jnp.uint32)
    vu = lax.bitcast_convert_type(
        v.astype(jnp.bfloat16).astype(jnp.float32), jnp.uint32)
    return lax.bitcast_convert_type(ku | (vu >> 16), jnp.float32)


def _unpack_k(w):
    wu = lax.bitcast_convert_type(w, jnp.uint32)
    return lax.bitcast_convert_type(wu & jnp.uint32(0xFFFF0000), jnp.float32)


def _unpack_v(w):
    wu = lax.bitcast_convert_type(w, jnp.uint32)
    return lax.bitcast_convert_type(wu << 16, jnp.float32)


def _qkv_kernel(x_ref, wq_ref, bq_ref, wk_ref, bk_ref, wv_ref,
                bv_ref, q_ref, kv_ref):
    xb = x_ref[0]

    def proj(w_ref, b_ref):
        return lax.dot_general(xb, w_ref[...], (((0,), (1,)), ((), ())),
                               preferred_element_type=jnp.float32) + b_ref[...]

    q_ref[0] = proj(wq_ref, bq_ref)
    kv_ref[0] = _pack_kv(proj(wk_ref, bk_ref), proj(wv_ref, bv_ref))


def _run_qkv(x, wq, bq, wk, bk, wv, bv):
    nt = _N // _TQ
    wspec = pl.BlockSpec((_C, _C), lambda b, i: (0, 0))
    bspec = pl.BlockSpec((1, _C), lambda b, i: (0, 0))
    q, kv = pl.pallas_call(
        _qkv_kernel,
        grid=(_B, nt),
        in_specs=[
            pl.BlockSpec((1, _C, _TQ), lambda b, i: (b, 0, i)),
            wspec, bspec, wspec, bspec, wspec, bspec,
        ],
        out_specs=[
            pl.BlockSpec((1, _TQ, _C), lambda b, i: (b, i, 0)),
            pl.BlockSpec((1, _TQ, _C), lambda b, i: (b, i, 0)),
        ],
        out_shape=[
            jax.ShapeDtypeStruct((_B, _N, _C), jnp.float32),
            jax.ShapeDtypeStruct((_B, _N, _C), jnp.float32),
        ],
    )(x, wq, bq.reshape(1, _C), wk, bk.reshape(1, _C), wv,
      bv.reshape(1, _C))
    return q.reshape(_B * _N, _C), kv.reshape(_B * _N, _C)


def _knn_kernel(pt_ref, pf_ref, pft_ref, idx_ref):
    b = pl.program_id(0)
    pt = pt_ref[0]
    pf = pf_ref[0]
    pft = pft_ref[0]
    sqt = (pt[:, 0:1] * pt[:, 0:1] + pt[:, 1:2] * pt[:, 1:2]) \
        + pt[:, 2:3] * pt[:, 2:3]
    sqf = (pft[0:1, :] * pft[0:1, :] + pft[1:2, :] * pft[1:2, :]) \
        + pft[2:3, :] * pft[2:3, :]
    dot = lax.dot_general(pt.astype(jnp.bfloat16), pf.astype(jnp.bfloat16),
                          (((1,), (1,)), ((), ())),
                          preferred_element_type=jnp.float32)
    d = (sqt + sqf) - 2.0 * dot

    iota = lax.broadcasted_iota(jnp.int32, d.shape, 1)
    big_i = jnp.int32(2 ** 30)
    inf = jnp.float32(3e38)
    cols = []
    for _ in range(_K):
        m = jnp.min(d, axis=1, keepdims=True)
        cand = jnp.where(d == m, iota, big_i)
        ij = jnp.min(cand, axis=1, keepdims=True)
        cols.append(ij)
        d = jnp.where(cand == ij, inf, d)
    idx_ref[0] = jnp.concatenate(cols, axis=1) + b * _N


def _run_knn(ppad3, pt8):
    nt = _N // _RK
    idx = pl.pallas_call(
        _knn_kernel,
        grid=(_B, nt),
        in_specs=[
            pl.BlockSpec((1, _RK, _PD), lambda b, i: (b, i, 0)),
            pl.BlockSpec((1, _N, _PD), lambda b, i: (b, 0, 0)),
            pl.BlockSpec((1, 8, _N), lambda b, i: (b, 0, 0)),
        ],
        out_specs=pl.BlockSpec((1, _RK, _K), lambda b, i: (b, i, 0)),
        out_shape=jax.ShapeDtypeStruct((_B, _N, _K), jnp.int32),
        compiler_params=pltpu.CompilerParams(
            dimension_semantics=("parallel", "parallel")),
    )(ppad3, ppad3, pt8)
    return idx.reshape(_CNT)


def _gather_rows(table, idx_flat, width, dtype=jnp.float32):
    mesh = plsc.VectorSubcoreMesh(core_axis_name="c", subcore_axis_name="s")

    @functools.partial(
        pl.kernel,
        mesh=mesh,
        out_type=jax.ShapeDtypeStruct((_CNT, width), dtype),
        scratch_types=[
            pltpu.VMEM((_CH,), jnp.int32),
            pltpu.VMEM((_CH, width), dtype),
            pltpu.SemaphoreType.DMA,
        ],
    )
    def gk(tab_hbm, idx_hbm, out_hbm, idx_v, row_v, s1):
        wid = lax.axis_index("s") * _NC + lax.axis_index("c")
        base = wid * _PER_W

        @pl.loop(0, _NCH)
        def _(c):
            off = base + c * _CH
            pltpu.sync_copy(idx_hbm.at[pl.ds(off, _CH)], idx_v)
            pltpu.async_copy(tab_hbm.at[idx_v], row_v, s1).wait()
            pltpu.sync_copy(row_v, out_hbm.at[pl.ds(off, _CH)])

    return gk(table, idx_flat)


def _rep_rows(t, k):
    tt, ll = t.shape
    return jnp.broadcast_to(t.reshape(tt, 1, ll), (tt, k, ll)).reshape(tt * k, ll)


def _p0_kernel(np_ref, pp_ref, s1_ref, s2_ref, rel_ref):
    rel = np_ref[:, 0:_PD] - _rep_rows(pp_ref[...], _K)
    rel_ref[...] = rel

    @pl.when(pl.program_id(0) == 0)
    def _():
        s1_ref[...] = jnp.zeros_like(s1_ref)
        s2_ref[...] = jnp.zeros_like(s2_ref)

    s1_ref[...] += jnp.sum(rel, axis=0, keepdims=True)
    s2_ref[...] += lax.dot_general(rel, rel, (((0,), (0,)), ((), ())),
                                   preferred_element_type=jnp.float32)


def _nr_from(rel, w1p, spe, tpe, w2pt, pecb):
    h = lax.dot_general(rel, w1p, (((1,), (1,)), ((), ())),
                        preferred_element_type=jnp.float32)
    hp = jnp.maximum(h * spe + tpe, 0.0)
    return lax.dot_general(hp, w2pt, (((1,), (0,)), ((), ())),
                           preferred_element_type=jnp.float32) + pecb


def _p1_kernel(nk_ref, rel_ref, q_ref, w1p_ref, spe_ref, tpe_ref,
               w2pt_ref, pecb_ref, s1_ref, s2_ref):
    nr = _nr_from(rel_ref[...], w1p_ref[...],
                  spe_ref[...], tpe_ref[...], w2pt_ref[...], pecb_ref[...])
    a = _rep_rows(q_ref[...], _K) - _unpack_k(nk_ref[...]) + nr

    @pl.when(pl.program_id(0) == 0)
    def _():
        s1_ref[...] = jnp.zeros_like(s1_ref)
        s2_ref[...] = jnp.zeros_like(s2_ref)

    s1_ref[...] += jnp.sum(a, axis=0, keepdims=True)
    s2_ref[...] += jnp.sum(a * a, axis=0, keepdims=True)


def _p2_kernel(nk_ref, rel_ref, q_ref, w1p_ref, spe_ref, tpe_ref,
               w2pt_ref, pecb_ref, s1a_ref, t1a_ref, aw1_ref,
               s1_ref, s2_ref):
    nr = _nr_from(rel_ref[...], w1p_ref[...],
                  spe_ref[...], tpe_ref[...], w2pt_ref[...], pecb_ref[...])
    a = _rep_rows(q_ref[...], _K) - _unpack_k(nk_ref[...]) + nr
    ap = jnp.maximum(a * s1a_ref[...] + t1a_ref[...], 0.0)
    a1 = lax.dot_general(ap, aw1_ref[...], (((1,), (1,)), ((), ())),
                         preferred_element_type=jnp.float32)

    @pl.when(pl.program_id(0) == 0)
    def _():
        s1_ref[...] = jnp.zeros_like(s1_ref)
        s2_ref[...] = jnp.zeros_like(s2_ref)

    s1_ref[...] += jnp.sum(a1, axis=0, keepdims=True)
    s2_ref[...] += jnp.sum(a1 * a1, axis=0, keepdims=True)


def _p3_kernel(kv_ref, rel_ref, q_ref, w1p_ref, spe_ref, tpe_ref,
               w2pt_ref, pecb_ref, s1a_ref, t1a_ref, aw1_ref,
               s2a_ref, t2a_ref, aw2_ref, acb_ref, exp_ref, y_ref):
    nr = _nr_from(rel_ref[...], w1p_ref[...],
                  spe_ref[...], tpe_ref[...], w2pt_ref[...], pecb_ref[...])
    kvb = kv_ref[...]
    nk = _unpack_k(kvb)
    nv = _unpack_v(kvb)
    a = _rep_rows(q_ref[...], _K) - nk + nr
    ap = jnp.maximum(a * s1a_ref[...] + t1a_ref[...], 0.0)
    a1 = lax.dot_general(ap, aw1_ref[...], (((1,), (1,)), ((), ())),
                         preferred_element_type=jnp.float32)
    a1p = jnp.maximum(a1 * s2a_ref[...] + t2a_ref[...], 0.0)
    a2 = lax.dot_general(a1p, aw2_ref[...], (((1,), (1,)), ((), ())),
                         preferred_element_type=jnp.float32) + acb_ref[...]
    a3 = a2.reshape(_TP, _K, _ATTN)
    mx = jnp.max(a3, axis=1, keepdims=True)
    e = jnp.exp(a3 - mx)
    w = e / jnp.sum(e, axis=1, keepdims=True)
    wf = w.reshape(_TP * _K, _ATTN)
    w128 = lax.dot_general(wf, exp_ref[...], (((1,), (0,)), ((), ())),
                           preferred_element_type=jnp.float32)
    contrib = w128 * (nv + nr)
    y_ref[...] = jnp.sum(contrib.reshape(_TP, _K, _C), axis=1)


def kernel(p, x, Wq, bq, Wk, bk, Wv, bv, pe_W1, pe_bn_g, pe_bn_b, pe_W2,
           pe_cb, at_bn1_g, at_bn1_b, at_W1, at_bn2_g, at_bn2_b, at_W2,
           at_cb):
    f = _f32
    p = f(p)
    x = f(x)

    pflat = p.reshape(_B * _N, 3)
    ppad = jnp.concatenate(
        [pflat, jnp.zeros((_B * _N, _PD - 3), jnp.float32)], axis=1)
    ppad3 = ppad.reshape(_B, _N, _PD)
    ppad128 = jnp.concatenate(
        [pflat, jnp.zeros((_B * _N, _C - 3), jnp.float32)], axis=1)

    pt8 = jnp.concatenate(
        [p.transpose(0, 2, 1), jnp.zeros((_B, 5, _N), jnp.float32)], axis=1)

    idx_flat = _run_knn(ppad3, pt8)
    p_g = _gather_rows(ppad128, idx_flat, _C)
    q, kvp = _run_qkv(x, f(Wq), f(bq), f(Wk), f(bk), f(Wv), f(bv))
    kv_g = _gather_rows(kvp, idx_flat, _C)

    grid = (_CNT // (_TP * _K),)
    np_spec = pl.BlockSpec((_TP * _K, _C), lambda i: (i, 0))
    pp_spec = pl.BlockSpec((_TP, _PD), lambda i: (i, 0))
    relc_spec = pl.BlockSpec((_TP * _K, _PD), lambda i: (i, 0))
    q_spec = pl.BlockSpec((_TP, _C), lambda i: (i, 0))
    nk_spec = pl.BlockSpec((_TP * _K, _C), lambda i: (i, 0))
    kv_spec = pl.BlockSpec((_TP * _K, _C), lambda i: (i, 0))

    def const_spec(shape):
        nd = len(shape)
        return pl.BlockSpec(shape, lambda i, _n=nd: (0,) * _n)

    acc_spec = const_spec

    s1r, s2r, rel_c = pl.pallas_call(
        _p0_kernel,
        grid=grid,
        in_specs=[np_spec, pp_spec],
        out_specs=[acc_spec((1, _PD)), acc_spec((_PD, _PD)), relc_spec],
        out_shape=[jax.ShapeDtypeStruct((1, _PD), jnp.float32),
                   jax.ShapeDtypeStruct((_PD, _PD), jnp.float32),
                   jax.ShapeDtypeStruct((_CNT, _PD), jnp.float32)],
    )(p_g, ppad)

    cnt = jnp.float32(_CNT)
    w1p = jnp.zeros((_PD, _PD), jnp.float32).at[0:3, 0:3].set(f(pe_W1))
    mean_rel = s1r / cnt
    cov = s2r / cnt - mean_rel.T @ mean_rel
    mh = mean_rel @ w1p.T
    vh = jnp.sum((w1p @ cov) * w1p, axis=1).reshape(1, _PD)
    g16 = jnp.zeros((1, _PD), jnp.float32).at[0, 0:3].set(f(pe_bn_g))
    b16 = jnp.zeros((1, _PD), jnp.float32).at[0, 0:3].set(f(pe_bn_b))
    spe = g16 * lax.rsqrt(vh + _EPS)
    tpe = b16 - mh * spe
    w2pt = jnp.zeros((_PD, _C), jnp.float32).at[0:3, :].set(f(pe_W2).T)
    pecb = f(pe_cb).reshape(1, _C)

    wconsts = [const_spec((_PD, _PD)), const_spec((1, _PD)),
               const_spec((1, _PD)), const_spec((_PD, _C)),
               const_spec((1, _C))]
    wvals = (w1p, spe, tpe, w2pt, pecb)

    s1a_s, s2a_s = pl.pallas_call(
        _p1_kernel,
        grid=grid,
        in_specs=[nk_spec, relc_spec, q_spec] + wconsts,
        out_specs=[acc_spec((1, _C)), acc_spec((1, _C))],
        out_shape=[jax.ShapeDtypeStruct((1, _C), jnp.float32),
                   jax.ShapeDtypeStruct((1, _C), jnp.float32)],
    )(kv_g, rel_c, q, *wvals)

    mean1 = s1a_s / cnt
    var1 = s2a_s / cnt - mean1 * mean1
    s1a = f(at_bn1_g).reshape(1, _C) * lax.rsqrt(var1 + _EPS)
    t1a = f(at_bn1_b).reshape(1, _C) - mean1 * s1a
    aw1 = f(at_W1)

    s1b_s, s2b_s = pl.pallas_call(
        _p2_kernel,
        grid=grid,
        in_specs=[nk_spec, relc_spec, q_spec] + wconsts +
                 [const_spec((1, _C)), const_spec((1, _C)),
                  const_spec((_ATTN, _C))],
        out_specs=[acc_spec((1, _ATTN)), acc_spec((1, _ATTN))],
        out_shape=[jax.ShapeDtypeStruct((1, _ATTN), jnp.float32),
                   jax.ShapeDtypeStruct((1, _ATTN), jnp.float32)],
    )(kv_g, rel_c, q, *wvals, s1a, t1a, aw1)

    mean2 = s1b_s / cnt
    var2 = s2b_s / cnt - mean2 * mean2
    s2a = f(at_bn2_g).reshape(1, _ATTN) * lax.rsqrt(var2 + _EPS)
    t2a = f(at_bn2_b).reshape(1, _ATTN) - mean2 * s2a
    aw2 = f(at_W2)
    acb = f(at_cb).reshape(1, _ATTN)
    expm = (jnp.arange(_ATTN)[:, None] ==
            (jnp.arange(_C)[None, :] // _SHARED)).astype(jnp.float32)

    y = pl.pallas_call(
        _p3_kernel,
        grid=grid,
        in_specs=[kv_spec, relc_spec, q_spec] + wconsts +
                 [const_spec((1, _C)), const_spec((1, _C)),
                  const_spec((_ATTN, _C)), const_spec((1, _ATTN)),
                  const_spec((1, _ATTN)), const_spec((_ATTN, _ATTN)),
                  const_spec((1, _ATTN)), const_spec((_ATTN, _C))],
        out_specs=pl.BlockSpec((_TP, _C), lambda i: (i, 0)),
        out_shape=jax.ShapeDtypeStruct((_B * _N, _C), jnp.float32),
    )(kv_g, rel_c, q, *wvals, s1a, t1a, aw1, s2a, t2a, aw2, acb, expm)

    return y.reshape(_B, _N, _C).transpose(0, 2, 1)
